# Initial kernel scaffold; baseline (speedup 1.0000x reference)
import functools

import jax
import jax.numpy as jnp
from jax import lax
from jax.experimental import pallas as pl
from jax.experimental.pallas import tpu as pltpu
from jax.experimental.pallas import tpu_sc as plsc

NC, NS, LANES = 2, 16, 16
NW = NC * NS
BATCH = 16384
EMB = 32
BPW = BATCH // NW
CHUNK = 128
NCHUNK = BPW // CHUNK
GROUPS = CHUNK // LANES

_mesh = plsc.VectorSubcoreMesh(core_axis_name="c", subcore_axis_name="s")


@functools.partial(
    pl.kernel,
    out_type=jax.ShapeDtypeStruct((2 * BATCH, EMB), jnp.float32),
    mesh=_mesh,
    scratch_types=[
        pltpu.VMEM((BPW, 5), jnp.int32),
        pltpu.VMEM((NCHUNK, CHUNK), jnp.int32),
        pltpu.VMEM((NCHUNK, CHUNK), jnp.int32),
        pltpu.VMEM((NCHUNK, CHUNK), jnp.int32),
        pltpu.VMEM((NCHUNK, CHUNK), jnp.int32),
        pltpu.VMEM((NCHUNK, CHUNK, EMB), jnp.float32),
        pltpu.VMEM((NCHUNK, CHUNK, EMB), jnp.float32),
        pltpu.SemaphoreType.DMA,
    ],
)
def _emb_kernel(fea_hbm, wcat_hbm, wbrand_hbm, out_hbm,
                fea_v, idx_cat, idx_brand, oidx_cat, oidx_brand,
                rows_cat, rows_brand, sem):
    wid = lax.axis_index("s") * NC + lax.axis_index("c")
    base = wid * BPW

    pltpu.sync_copy(fea_hbm.at[pl.ds(base, BPW)], fea_v)

    lanes = lax.iota(jnp.int32, LANES)
    col2 = jnp.full((LANES,), 2, jnp.int32)
    col3 = jnp.full((LANES,), 3, jnp.int32)
    for t in range(BPW // LANES):
        rows = t * LANES + lanes
        c, off = t // GROUPS, (t % GROUPS) * LANES
        idx_cat[c, pl.ds(off, LANES)] = plsc.load_gather(fea_v, [rows, col2])
        idx_brand[c, pl.ds(off, LANES)] = plsc.load_gather(fea_v, [rows, col3])
        orow = 2 * (base + rows)
        oidx_cat[c, pl.ds(off, LANES)] = orow
        oidx_brand[c, pl.ds(off, LANES)] = orow + 1

    gathers = []
    for c in range(NCHUNK):
        gathers.append(pltpu.make_async_copy(
            wcat_hbm.at[idx_cat.at[c]], rows_cat.at[c], sem))
        gathers.append(pltpu.make_async_copy(
            wbrand_hbm.at[idx_brand.at[c]], rows_brand.at[c], sem))
    for g in gathers:
        g.start()
    for g in gathers:
        g.wait()

    scatters = []
    for c in range(NCHUNK):
        scatters.append(pltpu.make_async_copy(
            rows_cat.at[c], out_hbm.at[oidx_cat.at[c]], sem))
        scatters.append(pltpu.make_async_copy(
            rows_brand.at[c], out_hbm.at[oidx_brand.at[c]], sem))
    for s in scatters:
        s.start()
    for s in scatters:
        s.wait()


def kernel(item_fea, W_iid, W_title, W_cat, W_brand, W_type):
    out = _emb_kernel(item_fea, W_cat, W_brand)
    return out.reshape(BATCH, 2 * EMB)


# speedup vs baseline: 2.1151x; 2.1151x over previous
import functools

import jax
import jax.numpy as jnp
from jax import lax
from jax.experimental import pallas as pl
from jax.experimental.pallas import tpu as pltpu
from jax.experimental.pallas import tpu_sc as plsc

NC, NS, LANES = 2, 16, 16
NW = NC * NS
BATCH = 16384
EMB = 32
BPW = BATCH // NW
CHUNK = 128
NCHUNK = BPW // CHUNK
GROUPS = CHUNK // LANES

_mesh = plsc.VectorSubcoreMesh(core_axis_name="c", subcore_axis_name="s")


@functools.partial(
    pl.kernel,
    out_type=jax.ShapeDtypeStruct((2 * BATCH, EMB), jnp.float32),
    mesh=_mesh,
    scratch_types=[
        pltpu.VMEM((BPW * 5,), jnp.int32),
        pltpu.VMEM((NCHUNK, CHUNK), jnp.int32),
        pltpu.VMEM((NCHUNK, CHUNK), jnp.int32),
        pltpu.VMEM((NCHUNK, CHUNK), jnp.int32),
        pltpu.VMEM((NCHUNK, CHUNK), jnp.int32),
        pltpu.VMEM((NCHUNK, CHUNK, EMB), jnp.float32),
        pltpu.VMEM((NCHUNK, CHUNK, EMB), jnp.float32),
        pltpu.SemaphoreType.DMA,
    ],
    compiler_params=pltpu.CompilerParams(
        needs_layout_passes=False, use_tc_tiling_on_sc=False),
)
def _emb_kernel(fea_hbm, wcat_hbm, wbrand_hbm, out_hbm,
                fea_v, idx_cat, idx_brand, oidx_cat, oidx_brand,
                rows_cat, rows_brand, sem):
    wid = lax.axis_index("s") * NC + lax.axis_index("c")
    base = wid * BPW

    pltpu.sync_copy(fea_hbm.at[pl.ds(base * 5, BPW * 5)], fea_v)

    lanes = lax.iota(jnp.int32, LANES)
    for t in range(BPW // LANES):
        rows = t * LANES + lanes
        c, off = t // GROUPS, (t % GROUPS) * LANES
        flat = 5 * rows
        idx_cat[c, pl.ds(off, LANES)] = plsc.load_gather(fea_v, [flat + 2])
        idx_brand[c, pl.ds(off, LANES)] = plsc.load_gather(fea_v, [flat + 3])
        orow = 2 * (base + rows)
        oidx_cat[c, pl.ds(off, LANES)] = orow
        oidx_brand[c, pl.ds(off, LANES)] = orow + 1

    gathers = []
    for c in range(NCHUNK):
        gathers.append(pltpu.make_async_copy(
            wcat_hbm.at[idx_cat.at[c]], rows_cat.at[c], sem))
        gathers.append(pltpu.make_async_copy(
            wbrand_hbm.at[idx_brand.at[c]], rows_brand.at[c], sem))
    for g in gathers:
        g.start()
    for g in gathers:
        g.wait()

    scatters = []
    for c in range(NCHUNK):
        scatters.append(pltpu.make_async_copy(
            rows_cat.at[c], out_hbm.at[oidx_cat.at[c]], sem))
        scatters.append(pltpu.make_async_copy(
            rows_brand.at[c], out_hbm.at[oidx_brand.at[c]], sem))
    for s in scatters:
        s.start()
    for s in scatters:
        s.wait()


def kernel(item_fea, W_iid, W_title, W_cat, W_brand, W_type):
    out = _emb_kernel(item_fea.reshape(BATCH * 5), W_cat, W_brand)
    return out.reshape(BATCH, 2 * EMB)
